# baseline (speedup 1.0000x reference)
import jax
import jax.numpy as jnp
from jax.experimental import pallas as pl
from jax.experimental.pallas import tpu as pltpu

_NCHUNK = 8


def _copy_body(src, dst, sems):
    rows = src.shape[0]
    chunk = rows // _NCHUNK
    for i in range(_NCHUNK):
        pltpu.make_async_copy(
            src.at[pl.ds(i * chunk, chunk)],
            dst.at[pl.ds(i * chunk, chunk)],
            sems.at[i],
        ).start()
    for i in range(_NCHUNK):
        pltpu.make_async_copy(
            src.at[pl.ds(i * chunk, chunk)],
            dst.at[pl.ds(i * chunk, chunk)],
            sems.at[i],
        ).wait()


def kernel(inputs, p_emb):
    return pl.pallas_call(
        _copy_body,
        out_shape=jax.ShapeDtypeStruct(p_emb.shape, p_emb.dtype),
        in_specs=[pl.BlockSpec(memory_space=pl.ANY)],
        out_specs=pl.BlockSpec(memory_space=pl.ANY),
        scratch_shapes=[pltpu.SemaphoreType.DMA((_NCHUNK,))],
    )(p_emb)


# speedup vs baseline: 16.1647x; 16.1647x over previous
import jax
import jax.numpy as jnp
from jax.experimental import pallas as pl
from jax.experimental.pallas import tpu as pltpu

_BLOCK_ROWS = 25000


def _copy_block(in_ref, out_ref):
    out_ref[...] = in_ref[...]


def kernel(inputs, p_emb):
    n, d = p_emb.shape
    grid = n // _BLOCK_ROWS
    return pl.pallas_call(
        _copy_block,
        grid=(grid,),
        in_specs=[pl.BlockSpec((_BLOCK_ROWS, d), lambda i: (i, 0))],
        out_specs=pl.BlockSpec((_BLOCK_ROWS, d), lambda i: (i, 0)),
        out_shape=jax.ShapeDtypeStruct(p_emb.shape, p_emb.dtype),
    )(p_emb)
